# baseline (speedup 1.0000x reference)
import functools

import jax
import jax.numpy as jnp
from jax import lax
from jax.experimental import pallas as pl
from jax.experimental.pallas import tpu as pltpu
from jax.experimental.pallas import tpu_sc as plsc


def _mm_body(x_ref, w_ref, b_ref, o_ref, *, relu):
    acc = jnp.dot(x_ref[...], w_ref[...], preferred_element_type=jnp.float32)
    acc = acc + b_ref[...]
    if relu:
        acc = jnp.maximum(acc, 0.0)
    o_ref[...] = acc


def _mm_bias(x, w, b, relu, tile_m):
    m, k = x.shape
    n = w.shape[1]
    return pl.pallas_call(
        functools.partial(_mm_body, relu=relu),
        grid=(m // tile_m,),
        in_specs=[
            pl.BlockSpec((tile_m, k), lambda i: (i, 0)),
            pl.BlockSpec((k, n), lambda i: (0, 0)),
            pl.BlockSpec((1, n), lambda i: (0, 0)),
        ],
        out_specs=pl.BlockSpec((tile_m, n), lambda i: (i, 0)),
        out_shape=jax.ShapeDtypeStruct((m, n), jnp.float32),
    )(x, w, b.reshape(1, n))


def _c1_body(w_ref, x_ref, b_ref, o_ref):
    acc = jnp.dot(w_ref[...], x_ref[0], preferred_element_type=jnp.float32)
    o_ref[0] = jnp.maximum(acc + b_ref[...], 0.0)


def _conv1_t(p1t, w1, b1, n_img):
    return pl.pallas_call(
        _c1_body,
        grid=(n_img,),
        in_specs=[
            pl.BlockSpec((64, 48), lambda i: (0, 0)),
            pl.BlockSpec((1, 48, 12544), lambda i: (i, 0, 0)),
            pl.BlockSpec((64, 1), lambda i: (0, 0)),
        ],
        out_specs=pl.BlockSpec((1, 64, 12544), lambda i: (i, 0, 0)),
        out_shape=jax.ShapeDtypeStruct((n_img, 64, 12544), jnp.float32),
    )(w1, p1t, b1.reshape(64, 1))


def _tmm_body(w_ref, x_ref, o_ref):
    o_ref[0] = jnp.dot(w_ref[...], x_ref[0],
                       preferred_element_type=jnp.float32)


def _tmm(w, x3, n_img, m, nn):
    return pl.pallas_call(
        _tmm_body,
        grid=(n_img,),
        in_specs=[
            pl.BlockSpec(w.shape, lambda i: (0, 0)),
            pl.BlockSpec((1, x3.shape[1], nn), lambda i: (i, 0, 0)),
        ],
        out_specs=pl.BlockSpec((1, m, nn), lambda i: (i, 0, 0)),
        out_shape=jax.ShapeDtypeStruct((n_img, m, nn), jnp.float32),
    )(w, x3)


def _c2vq_body(y_ref, cbt_ref, b_ref, z_ref, idx_ref):
    y = y_ref[0]
    acc = b_ref[...]
    for a in range(2):
        for b in range(2):
            t = a * 2 + b
            s0 = a * 64 + b
            acc = acc + y[t * 64:(t + 1) * 64, s0:s0 + 3584]
    z_ref[0] = acc
    cbt = cbt_ref[...]
    cn = jnp.sum(cbt * cbt, axis=0, keepdims=True)
    zc = jax.lax.dot_general(acc, cbt, (((0,), (0,)), ((), ())),
                             preferred_element_type=jnp.float32)
    s = cn - 2.0 * zc
    idx_ref[0] = jnp.argmin(s, axis=1).astype(jnp.int32).reshape(1, 3584)


def _conv2_vq(y2, cbt, b2, n_img):
    z, idx = pl.pallas_call(
        _c2vq_body,
        grid=(n_img,),
        in_specs=[
            pl.BlockSpec((1, 256, 3712), lambda i: (i, 0, 0)),
            pl.BlockSpec((64, 1024), lambda i: (0, 0)),
            pl.BlockSpec((64, 1), lambda i: (0, 0)),
        ],
        out_specs=[
            pl.BlockSpec((1, 64, 3584), lambda i: (i, 0, 0)),
            pl.BlockSpec((1, 1, 3584), lambda i: (i, 0, 0)),
        ],
        out_shape=[
            jax.ShapeDtypeStruct((n_img, 64, 3584), jnp.float32),
            jax.ShapeDtypeStruct((n_img, 1, 3584), jnp.int32),
        ],
    )(y2, cbt, b2.reshape(64, 1))
    return z, idx.reshape(n_img, 3584)


def _t1_body(y_ref, b_ref, o_ref):
    yf = y_ref[0]
    bias = b_ref[...].reshape(1, 1, -1)
    for r in range(2):
        for s in range(2):
            t = r * 2 + s
            ph = yf[r:r + 56, s:s + 56, t * 64:(t + 1) * 64] + bias
            o_ref[0, t] = jnp.maximum(ph, 0.0)


def _t1_phases(y4, b3, n_img):
    y4v = y4.reshape(n_img, 57, 64, 256)
    return pl.pallas_call(
        _t1_body,
        grid=(n_img,),
        in_specs=[
            pl.BlockSpec((1, 57, 64, 256), lambda i: (i, 0, 0, 0)),
            pl.BlockSpec((1, 64), lambda i: (0, 0)),
        ],
        out_specs=pl.BlockSpec((1, 4, 56, 56, 64), lambda i: (i, 0, 0, 0, 0)),
        out_shape=jax.ShapeDtypeStruct((n_img, 4, 56, 56, 64), jnp.float32),
    )(y4v, b3.reshape(1, 64))


def _t2_body(h1_ref, h2_ref, w9_ref, b_ref, o_ref):
    x = jnp.concatenate([h1_ref[0], h2_ref[0]], axis=0).reshape(768, 64)
    acc = jnp.broadcast_to(b_ref[...], (384, 16))
    for dr in range(3):
        for dc in range(3):
            s0 = dr * 128 + dc
            acc = acc + jnp.dot(x[s0:s0 + 384, :], w9_ref[dr * 3 + dc],
                                preferred_element_type=jnp.float32)
    o_ref[0, 0] = acc


def _conv_t2(hp, w9, b16, n_img):
    return pl.pallas_call(
        _t2_body,
        grid=(n_img, 38),
        in_specs=[
            pl.BlockSpec((1, 3, 128, 64), lambda i, t: (i, t, 0, 0)),
            pl.BlockSpec((1, 3, 128, 64), lambda i, t: (i, t + 1, 0, 0)),
            pl.BlockSpec((9, 64, 16), lambda i, t: (0, 0, 0)),
            pl.BlockSpec((1, 16), lambda i, t: (0, 0)),
        ],
        out_specs=pl.BlockSpec((1, 1, 384, 16), lambda i, t: (i, t, 0, 0)),
        out_shape=jax.ShapeDtypeStruct((n_img, 38, 384, 16), jnp.float32),
    )(hp, hp, w9, b16.reshape(1, 16)).reshape(n_img, 38, 3, 128, 16).reshape(
        n_img, 114, 128, 16)


def _sc_gather(table, idx):
    nw, b_per_w = idx.shape
    b = nw * b_per_w
    k, d = table.shape
    chunk = b_per_w // 2
    groups = chunk // 16
    info = plsc.get_sparse_core_info()
    mesh = plsc.VectorSubcoreMesh(core_axis_name="c", subcore_axis_name="s")

    @functools.partial(
        pl.kernel,
        mesh=mesh,
        compiler_params=pltpu.CompilerParams(needs_layout_passes=False),
        out_type=jax.ShapeDtypeStruct((b * d,), jnp.float32),
        scratch_types=[
            pltpu.VMEM((k * d,), jnp.float32),
            pltpu.VMEM((b_per_w,), jnp.int32),
            pltpu.VMEM((chunk * d,), jnp.float32),
        ],
    )
    def gather_kernel(table_hbm, idx_hbm, out_hbm, table_v, idx_v, rows_v):
        wid = lax.axis_index("s") * info.num_cores + lax.axis_index("c")
        base = wid * b_per_w
        pltpu.sync_copy(table_hbm, table_v)
        pltpu.sync_copy(idx_hbm.at[wid], idx_v)
        lane = lax.iota(jnp.int32, 16)
        for ch in range(2):

            def body(g, carry):
                rows16 = idx_v[pl.ds(ch * chunk + g * 16, 16)] * d
                loc16 = (g * 16 + lane) * d
                for c in range(d):
                    vals = plsc.load_gather(table_v, [rows16 + c])
                    plsc.store_scatter(rows_v, [loc16 + c], vals)
                return carry

            lax.fori_loop(0, groups, body, 0)
            pltpu.sync_copy(
                rows_v,
                out_hbm.at[pl.ds((base + ch * chunk) * d, chunk * d)])

    return gather_kernel(table.reshape(k * d), idx).reshape(b, d)


def _w1_mat(enc_w1):
    w = enc_w1.reshape(64, 3, 2, 2, 2, 2)
    return w.transpose(2, 4, 3, 5, 1, 0).reshape(48, 64)


def _w2_mat(enc_w2):
    w = enc_w2.reshape(64, 64, 2, 2, 2, 2)
    return w.transpose(3, 5, 1, 2, 4, 0).reshape(256, 256)


def _w4_mat(dec_w1):
    w = dec_w1.reshape(64, 64, 2, 2, 2, 2)
    return w.transpose(2, 4, 1, 3, 5, 0).reshape(256, 256)


def _w9_mat(dec_w2):
    w9 = jnp.zeros((9, 64, 2, 2, 4), jnp.float32)
    for dr in range(3):
        for r in range(2):
            if not 0 <= dr - r <= 1:
                continue
            for dc in range(3):
                for s in range(2):
                    if not 0 <= dc - s <= 1:
                        continue
                    w9 = w9.at[dr * 3 + dc, :, r, s, :3].set(
                        dec_w2[:, :, 2 * dr - r, 2 * dc - s].T)
    return w9.reshape(9, 64, 16)


def kernel(x, x_cond, y, enc_w1, enc_b1, enc_w2, enc_b2, codebook,
           dec_w1, dec_b1, dec_w2, dec_b2):
    n = x.shape[0]
    d = codebook.shape[1]

    xp = jnp.pad(x, ((0, 0), (0, 0), (1, 1), (1, 1)))
    p1t = jnp.stack(
        [xp[:, c, ki:ki + 223:2, kj:kj + 223:2]
         for c in range(3) for ki in range(4) for kj in range(4)],
        axis=1)
    h1t = _conv1_t(p1t.reshape(n, 48, 12544), enc_w1.reshape(64, 48),
                   enc_b1, n)

    h1v = h1t.reshape(n, 64, 112, 112)
    h1p = jnp.pad(h1v, ((0, 0), (0, 0), (1, 1), (1, 1)))
    p2t = jnp.concatenate(
        [h1p[:, :, r::2, s::2] for r in range(2) for s in range(2)],
        axis=1)
    p2t = jnp.pad(p2t, ((0, 0), (0, 0), (0, 1), (0, 7)))
    y2 = _tmm(_w2_mat(enc_w2).T, p2t.reshape(n, 256, 3712),
              n, 256, 3712)
    z, idx = _conv2_vq(y2, codebook.T, enc_b2, n)

    idx_valid = idx.reshape(n, 56, 64)[:, :, :56]
    q = _sc_gather(codebook,
                   idx_valid.reshape(32, (n * 3136) // 32))

    qv = q.reshape(n, 56, 56, d)
    qp = jnp.pad(qv, ((0, 0), (1, 1), (1, 1), (0, 0)))
    p4 = jnp.concatenate(
        [qp[:, a:a + 57, b:b + 57, :] for a in range(2) for b in range(2)],
        axis=-1)
    p4 = jnp.pad(p4, ((0, 0), (0, 0), (0, 7), (0, 0)))
    y4 = _mm_bias(p4.reshape(n * 57 * 64, 256), _w4_mat(dec_w1),
                  jnp.zeros((256,), jnp.float32), False, 1024)
    o4 = _t1_phases(y4, dec_b1, n)

    hdec = (o4.reshape(n, 2, 2, 56, 56, 64)
            .transpose(0, 3, 1, 4, 2, 5).reshape(n, 112, 112, 64))
    hp = jnp.pad(hdec, ((0, 0), (1, 7), (1, 15), (0, 0)))
    b16 = jnp.tile(jnp.pad(dec_b2, (0, 1)), 4)
    o5 = _conv_t2(hp, _w9_mat(dec_w2), b16, n)

    x_hat = (o5.reshape(n, 114, 128, 2, 2, 4)[:, :112, :112, :, :, :3]
             .transpose(0, 5, 1, 3, 2, 4).reshape(n, 3, 224, 224))
    latent = z.reshape(n, 64, 56, 64)[:, :, :, :56]
    quantized = qv.transpose(0, 3, 1, 2)
    emb_idx = idx_valid
    return (x_hat, quantized, latent, emb_idx)


# speedup vs baseline: 1.2405x; 1.2405x over previous
import functools

import jax
import jax.numpy as jnp
from jax import lax
from jax.experimental import pallas as pl
from jax.experimental.pallas import tpu as pltpu
from jax.experimental.pallas import tpu_sc as plsc


def _mm_body(x_ref, w_ref, b_ref, o_ref, *, relu):
    acc = jnp.dot(x_ref[...], w_ref[...], preferred_element_type=jnp.float32)
    acc = acc + b_ref[...]
    if relu:
        acc = jnp.maximum(acc, 0.0)
    o_ref[...] = acc


def _mm_bias(x, w, b, relu, tile_m):
    m, k = x.shape
    n = w.shape[1]
    return pl.pallas_call(
        functools.partial(_mm_body, relu=relu),
        grid=(m // tile_m,),
        in_specs=[
            pl.BlockSpec((tile_m, k), lambda i: (i, 0)),
            pl.BlockSpec((k, n), lambda i: (0, 0)),
            pl.BlockSpec((1, n), lambda i: (0, 0)),
        ],
        out_specs=pl.BlockSpec((tile_m, n), lambda i: (i, 0)),
        out_shape=jax.ShapeDtypeStruct((m, n), jnp.float32),
    )(x, w, b.reshape(1, n))


def _c1_body(w_ref, x_ref, b_ref, o_ref):
    acc = jnp.dot(w_ref[...], x_ref[0], preferred_element_type=jnp.float32)
    o_ref[0] = jnp.maximum(acc + b_ref[...], 0.0)


def _conv1_t(p1t, w1, b1, n_img):
    return pl.pallas_call(
        _c1_body,
        grid=(n_img,),
        in_specs=[
            pl.BlockSpec((64, 48), lambda i: (0, 0)),
            pl.BlockSpec((1, 48, 12544), lambda i: (i, 0, 0)),
            pl.BlockSpec((64, 1), lambda i: (0, 0)),
        ],
        out_specs=pl.BlockSpec((1, 64, 12544), lambda i: (i, 0, 0)),
        out_shape=jax.ShapeDtypeStruct((n_img, 64, 12544), jnp.float32),
    )(w1, p1t, b1.reshape(64, 1))


def _tmm_body(w_ref, x_ref, o_ref):
    o_ref[0] = jnp.dot(w_ref[...], x_ref[0],
                       preferred_element_type=jnp.float32)


def _tmm(w, x3, n_img, m, nn):
    return pl.pallas_call(
        _tmm_body,
        grid=(n_img,),
        in_specs=[
            pl.BlockSpec(w.shape, lambda i: (0, 0)),
            pl.BlockSpec((1, x3.shape[1], nn), lambda i: (i, 0, 0)),
        ],
        out_specs=pl.BlockSpec((1, m, nn), lambda i: (i, 0, 0)),
        out_shape=jax.ShapeDtypeStruct((n_img, m, nn), jnp.float32),
    )(w, x3)


def _c2vq_body(y_ref, cbt_ref, b_ref, z_ref, idx_ref):
    yf = y_ref[0]
    acc = b_ref[...].reshape(1, 1, -1)
    for a in range(2):
        for b in range(2):
            t = a * 2 + b
            acc = acc + yf[a:a + 56, b:b + 56, t * 64:(t + 1) * 64]
    z_ref[0] = acc
    zf = acc.reshape(3136, 64)
    cbt = cbt_ref[...]
    cn = jnp.sum(cbt * cbt, axis=0, keepdims=True)
    s = cn - 2.0 * jnp.dot(zf, cbt, preferred_element_type=jnp.float32)
    idx_ref[0] = jnp.argmin(s, axis=1).astype(jnp.int32).reshape(1, 3136)


def _conv2_vq(y2, cbt, b2, n_img):
    y2v = y2.reshape(n_img, 57, 64, 256)
    z, idx = pl.pallas_call(
        _c2vq_body,
        grid=(n_img,),
        in_specs=[
            pl.BlockSpec((1, 57, 64, 256), lambda i: (i, 0, 0, 0)),
            pl.BlockSpec((64, 1024), lambda i: (0, 0)),
            pl.BlockSpec((1, 64), lambda i: (0, 0)),
        ],
        out_specs=[
            pl.BlockSpec((1, 56, 56, 64), lambda i: (i, 0, 0, 0)),
            pl.BlockSpec((1, 1, 3136), lambda i: (i, 0, 0)),
        ],
        out_shape=[
            jax.ShapeDtypeStruct((n_img, 56, 56, 64), jnp.float32),
            jax.ShapeDtypeStruct((n_img, 1, 3136), jnp.int32),
        ],
    )(y2v, cbt, b2.reshape(1, 64))
    return z, idx.reshape(n_img * 3136)


def _t1_body(y_ref, b_ref, o_ref):
    yf = y_ref[0]
    bias = b_ref[...].reshape(1, 1, -1)
    for r in range(2):
        for s in range(2):
            t = r * 2 + s
            ph = yf[r:r + 56, s:s + 56, t * 64:(t + 1) * 64] + bias
            o_ref[0, t] = jnp.maximum(ph, 0.0)


def _t1_phases(y4, b3, n_img):
    y4v = y4.reshape(n_img, 57, 64, 256)
    return pl.pallas_call(
        _t1_body,
        grid=(n_img,),
        in_specs=[
            pl.BlockSpec((1, 57, 64, 256), lambda i: (i, 0, 0, 0)),
            pl.BlockSpec((1, 64), lambda i: (0, 0)),
        ],
        out_specs=pl.BlockSpec((1, 4, 56, 56, 64), lambda i: (i, 0, 0, 0, 0)),
        out_shape=jax.ShapeDtypeStruct((n_img, 4, 56, 56, 64), jnp.float32),
    )(y4v, b3.reshape(1, 64))


def _t2_body(h1_ref, h2_ref, w9_ref, b_ref, o_ref):
    x = jnp.concatenate([h1_ref[0], h2_ref[0]], axis=0).reshape(768, 64)
    acc = jnp.broadcast_to(b_ref[...], (384, 16))
    for dr in range(3):
        for dc in range(3):
            s0 = dr * 128 + dc
            acc = acc + jnp.dot(x[s0:s0 + 384, :], w9_ref[dr * 3 + dc],
                                preferred_element_type=jnp.float32)
    o_ref[0, 0] = acc


def _conv_t2(hp, w9, b16, n_img):
    return pl.pallas_call(
        _t2_body,
        grid=(n_img, 38),
        in_specs=[
            pl.BlockSpec((1, 3, 128, 64), lambda i, t: (i, t, 0, 0)),
            pl.BlockSpec((1, 3, 128, 64), lambda i, t: (i, t + 1, 0, 0)),
            pl.BlockSpec((9, 64, 16), lambda i, t: (0, 0, 0)),
            pl.BlockSpec((1, 16), lambda i, t: (0, 0)),
        ],
        out_specs=pl.BlockSpec((1, 1, 384, 16), lambda i, t: (i, t, 0, 0)),
        out_shape=jax.ShapeDtypeStruct((n_img, 38, 384, 16), jnp.float32),
    )(hp, hp, w9, b16.reshape(1, 16)).reshape(n_img, 38, 3, 128, 16).reshape(
        n_img, 114, 128, 16)


def _sc_gather(table, idx):
    nw, b_per_w = idx.shape
    b = nw * b_per_w
    k, d = table.shape
    chunk = b_per_w // 2
    groups = chunk // 16
    info = plsc.get_sparse_core_info()
    mesh = plsc.VectorSubcoreMesh(core_axis_name="c", subcore_axis_name="s")

    @functools.partial(
        pl.kernel,
        mesh=mesh,
        compiler_params=pltpu.CompilerParams(needs_layout_passes=False),
        out_type=jax.ShapeDtypeStruct((b * d,), jnp.float32),
        scratch_types=[
            pltpu.VMEM((k * d,), jnp.float32),
            pltpu.VMEM((b_per_w,), jnp.int32),
            pltpu.VMEM((chunk * d,), jnp.float32),
        ],
    )
    def gather_kernel(table_hbm, idx_hbm, out_hbm, table_v, idx_v, rows_v):
        wid = lax.axis_index("s") * info.num_cores + lax.axis_index("c")
        base = wid * b_per_w
        pltpu.sync_copy(table_hbm, table_v)
        pltpu.sync_copy(idx_hbm.at[wid], idx_v)
        lane = lax.iota(jnp.int32, 16)
        for ch in range(2):

            def body(g, carry):
                rows16 = idx_v[pl.ds(ch * chunk + g * 16, 16)] * d
                loc16 = (g * 16 + lane) * d
                for c in range(d):
                    vals = plsc.load_gather(table_v, [rows16 + c])
                    plsc.store_scatter(rows_v, [loc16 + c], vals)
                return carry

            lax.fori_loop(0, groups, body, 0)
            pltpu.sync_copy(
                rows_v,
                out_hbm.at[pl.ds((base + ch * chunk) * d, chunk * d)])

    return gather_kernel(table.reshape(k * d), idx).reshape(b, d)


def _w1_mat(enc_w1):
    w = enc_w1.reshape(64, 3, 2, 2, 2, 2)
    return w.transpose(2, 4, 3, 5, 1, 0).reshape(48, 64)


def _w2_mat(enc_w2):
    w = enc_w2.reshape(64, 64, 2, 2, 2, 2)
    return w.transpose(3, 5, 1, 2, 4, 0).reshape(256, 256)


def _w4_mat(dec_w1):
    w = dec_w1.reshape(64, 64, 2, 2, 2, 2)
    return w.transpose(2, 4, 1, 3, 5, 0).reshape(256, 256)


def _w9_mat(dec_w2):
    w9 = jnp.zeros((9, 64, 2, 2, 4), jnp.float32)
    for dr in range(3):
        for r in range(2):
            if not 0 <= dr - r <= 1:
                continue
            for dc in range(3):
                for s in range(2):
                    if not 0 <= dc - s <= 1:
                        continue
                    w9 = w9.at[dr * 3 + dc, :, r, s, :3].set(
                        dec_w2[:, :, 2 * dr - r, 2 * dc - s].T)
    return w9.reshape(9, 64, 16)


def kernel(x, x_cond, y, enc_w1, enc_b1, enc_w2, enc_b2, codebook,
           dec_w1, dec_b1, dec_w2, dec_b2):
    n = x.shape[0]
    d = codebook.shape[1]

    xn = x.transpose(0, 2, 3, 1)
    xp = jnp.pad(xn, ((0, 0), (1, 1), (1, 1), (0, 0)))
    xph = jnp.concatenate(
        [xp[:, r::2, s::2, :] for r in range(2) for s in range(2)],
        axis=-1)
    p1 = jnp.concatenate(
        [xph[:, a:a + 112, b:b + 112, :] for a in range(2) for b in range(2)],
        axis=-1)
    h1 = _mm_bias(p1.reshape(n * 112 * 112, 48), _w1_mat(enc_w1),
                  enc_b1, True, 2048)

    h1v = h1.reshape(n, 112, 112, 64)
    h1p = jnp.pad(h1v, ((0, 0), (1, 1), (1, 1), (0, 0)))
    p2 = jnp.concatenate(
        [h1p[:, r::2, s::2, :] for r in range(2) for s in range(2)],
        axis=-1)
    p2 = jnp.pad(p2, ((0, 0), (0, 0), (0, 7), (0, 0)))
    y2 = _mm_bias(p2.reshape(n * 57 * 64, 256), _w2_mat(enc_w2),
                  jnp.zeros((256,), jnp.float32), False, 1024)
    z, idx = _conv2_vq(y2, codebook.T, enc_b2, n)

    q = _sc_gather(codebook, idx.reshape(32, (n * 3136) // 32))

    qv = q.reshape(n, 56, 56, d)
    qp = jnp.pad(qv, ((0, 0), (1, 1), (1, 1), (0, 0)))
    p4 = jnp.concatenate(
        [qp[:, a:a + 57, b:b + 57, :] for a in range(2) for b in range(2)],
        axis=-1)
    p4 = jnp.pad(p4, ((0, 0), (0, 0), (0, 7), (0, 0)))
    y4 = _mm_bias(p4.reshape(n * 57 * 64, 256), _w4_mat(dec_w1),
                  jnp.zeros((256,), jnp.float32), False, 1024)
    o4 = _t1_phases(y4, dec_b1, n)

    hdec = (o4.reshape(n, 2, 2, 56, 56, 64)
            .transpose(0, 3, 1, 4, 2, 5).reshape(n, 112, 112, 64))
    hp = jnp.pad(hdec, ((0, 0), (1, 7), (1, 15), (0, 0)))
    b16 = jnp.tile(jnp.pad(dec_b2, (0, 1)), 4)
    o5 = _conv_t2(hp, _w9_mat(dec_w2), b16, n)

    x_hat = (o5.reshape(n, 114, 128, 2, 2, 4)[:, :112, :112, :, :, :3]
             .transpose(0, 5, 1, 3, 2, 4).reshape(n, 3, 224, 224))
    latent = z.transpose(0, 3, 1, 2)
    quantized = qv.transpose(0, 3, 1, 2)
    emb_idx = idx.reshape(n, 56, 56)
    return (x_hat, quantized, latent, emb_idx)
